# baseline (speedup 1.0000x reference)
import jax
import jax.numpy as jnp
from jax.experimental import pallas as pl
from jax.experimental.pallas import tpu as pltpu


def _body(out_ref, idx_ref, w2_ref, b2_ref, w3_ref, b3_ref, o_ref,
          stash, t3_s, msg_s):
    i = pl.program_id(0)
    B = stash.shape[0]

    @pl.when(i < B)
    def _phase1():
        x = out_ref[0]
        stash[i] = x
        s = jnp.sum(x, axis=0, keepdims=True)
        m = jax.lax.dot_general(
            s, w2_ref[...], (((1,), (0,)), ((), ())),
            preferred_element_type=jnp.float32) + b2_ref[...]
        t3_s[pl.ds(i, 1), :] = jax.lax.dot_general(
            m, w3_ref[...], (((1,), (0,)), ((), ())),
            preferred_element_type=jnp.float32)

    @pl.when(i == B)
    def _gather():
        G = msg_s.shape[0]
        iota = jax.lax.broadcasted_iota(jnp.int32, (G, B), 1)
        onehot = (idx_ref[...] == iota).astype(jnp.float32)
        msg_s[...] = jax.lax.dot_general(
            onehot, t3_s[...], (((1,), (0,)), ((), ())),
            preferred_element_type=jnp.float32) + b3_ref[...]

    @pl.when(i >= B)
    def _phase2():
        b = i - B
        o_ref[0] = jax.lax.dot_general(
            stash[b], w3_ref[...], (((1,), (0,)), ((), ())),
            preferred_element_type=jnp.float32) + msg_s[...]


def kernel(out, coords_neighbors_idx, n_batch, n_grid, n_ao, W2, b2, W3, b3):
    B, G, A = out.shape
    idx2d = coords_neighbors_idx.astype(jnp.int32).reshape(G, 1)
    return pl.pallas_call(
        _body,
        grid=(2 * B,),
        in_specs=[
            pl.BlockSpec((1, G, A), lambda i: (jnp.minimum(i, B - 1), 0, 0)),
            pl.BlockSpec((G, 1), lambda i: (0, 0)),
            pl.BlockSpec((A, A), lambda i: (0, 0)),
            pl.BlockSpec((1, A), lambda i: (0, 0)),
            pl.BlockSpec((A, A), lambda i: (0, 0)),
            pl.BlockSpec((1, A), lambda i: (0, 0)),
        ],
        out_specs=pl.BlockSpec((1, G, A), lambda i: (jnp.maximum(i - B, 0), 0, 0)),
        out_shape=jax.ShapeDtypeStruct((B, G, A), jnp.float32),
        scratch_shapes=[
            pltpu.VMEM((B, G, A), jnp.float32),
            pltpu.VMEM((B, A), jnp.float32),
            pltpu.VMEM((G, A), jnp.float32),
        ],
    )(out, idx2d, W2, b2.reshape(1, A), W3, b3.reshape(1, A))


# speedup vs baseline: 1.2501x; 1.2501x over previous
import jax
import jax.numpy as jnp
from jax.experimental import pallas as pl
from jax.experimental.pallas import tpu as pltpu

_CPB = 2


def _body(in_hbm, idx_ref, w2_ref, b2_ref, w3_ref, b3_ref, o_hbm,
          vbuf, obuf, t3_s, sin, sout):
    B, G, A = in_hbm.shape
    half = G // _CPB

    def in_copy(b, j):
        return pltpu.make_async_copy(
            in_hbm.at[b, pl.ds(j * half, half)],
            vbuf.at[b, pl.ds(j * half, half)],
            sin.at[b * _CPB + j])

    for b in range(B):
        for j in range(_CPB):
            in_copy(b, j).start()

    for b in range(B):
        for j in range(_CPB):
            in_copy(b, j).wait()
        s = jnp.sum(vbuf[b], axis=0, keepdims=True)
        m = jax.lax.dot_general(
            s, w2_ref[...], (((1,), (0,)), ((), ())),
            preferred_element_type=jnp.float32) + b2_ref[...]
        t3_s[pl.ds(b, 1), :] = jax.lax.dot_general(
            m, w3_ref[...], (((1,), (0,)), ((), ())),
            preferred_element_type=jnp.float32)

    iota = jax.lax.broadcasted_iota(jnp.int32, (G, B), 1)
    onehot = (idx_ref[...] == iota).astype(jnp.float32)
    msg = jax.lax.dot_general(
        onehot, t3_s[...], (((1,), (0,)), ((), ())),
        preferred_element_type=jnp.float32) + b3_ref[...]

    for b in range(B):
        obuf[b] = jax.lax.dot_general(
            vbuf[b], w3_ref[...], (((1,), (0,)), ((), ())),
            preferred_element_type=jnp.float32) + msg
        pltpu.make_async_copy(obuf.at[b], o_hbm.at[b], sout.at[b]).start()
    for b in range(B):
        pltpu.make_async_copy(obuf.at[b], o_hbm.at[b], sout.at[b]).wait()


def kernel(out, coords_neighbors_idx, n_batch, n_grid, n_ao, W2, b2, W3, b3):
    B, G, A = out.shape
    idx2d = coords_neighbors_idx.astype(jnp.int32).reshape(G, 1)
    return pl.pallas_call(
        _body,
        in_specs=[
            pl.BlockSpec(memory_space=pltpu.MemorySpace.HBM),
            pl.BlockSpec(memory_space=pltpu.MemorySpace.VMEM),
            pl.BlockSpec(memory_space=pltpu.MemorySpace.VMEM),
            pl.BlockSpec(memory_space=pltpu.MemorySpace.VMEM),
            pl.BlockSpec(memory_space=pltpu.MemorySpace.VMEM),
            pl.BlockSpec(memory_space=pltpu.MemorySpace.VMEM),
        ],
        out_specs=pl.BlockSpec(memory_space=pltpu.MemorySpace.HBM),
        out_shape=jax.ShapeDtypeStruct((B, G, A), jnp.float32),
        scratch_shapes=[
            pltpu.VMEM((B, G, A), jnp.float32),
            pltpu.VMEM((B, G, A), jnp.float32),
            pltpu.VMEM((B, A), jnp.float32),
            pltpu.SemaphoreType.DMA((B * _CPB,)),
            pltpu.SemaphoreType.DMA((B,)),
        ],
    )(out, idx2d, W2, b2.reshape(1, A), W3, b3.reshape(1, A))
